# baseline (speedup 1.0000x reference)
import dataclasses
import functools

import jax
import jax.numpy as jnp
import numpy as np
from jax.experimental import pallas as pl
from jax.experimental.pallas import tpu as pltpu
from jax.experimental.pallas import tpu_sc as plsc

_SIGN = np.int32(-2**31)
_LOW31 = np.int32(2**31 - 1)
_BITS = [np.int32(-2**31)] + [np.int32(1 << i) for i in range(30, -1, -1)]


def _float_keys(x):
    b = jax.lax.bitcast_convert_type(x, jnp.int32)
    return jnp.where(b >= 0, b, b ^ _LOW31)


def _pass1_kernel(z_ref, w_ref, bphi_ref, keys_ref, *, s_blk):
    b, c, _ = z_ref.shape
    acc = z_ref[:, 0:8, :] * w_ref[:, 0:8, :]
    for r in range(1, c // 8):
        acc = acc + z_ref[:, 8 * r:8 * (r + 1), :] * w_ref[:, 8 * r:8 * (r + 1), :]
    xv = jnp.sum(acc, axis=1) + bphi_ref[...]
    keys_ref[...] = _float_keys(xv)


def _sc_select(keys, k_rank, hw, n_lanes=16):
    b = keys.shape[0]
    mesh = plsc.VectorSubcoreMesh(core_axis_name="c", subcore_axis_name="s")
    cp = pltpu.CompilerParams()
    if "needs_layout_passes" in pltpu.CompilerParams.__dataclass_fields__:
        cp = dataclasses.replace(cp, needs_layout_passes=False)

    @pl.kernel(out_type=jax.ShapeDtypeStruct((b, n_lanes), jnp.int32),
               mesh=mesh,
               scratch_types=[pltpu.VMEM((hw,), jnp.int32),
                              pltpu.VMEM((n_lanes,), jnp.int32),
                              pltpu.VMEM((n_lanes,), jnp.int32),
                              pltpu.SemaphoreType.DMA],
               compiler_params=cp)
    def select(keys_hbm, o_hbm, buf, acc_ref, res_ref, sem):
        core = jax.lax.axis_index("c")
        sub = jax.lax.axis_index("s")
        gid = core * 16 + sub

        @pl.when(gid < b)
        def _():
            pltpu.async_copy(keys_hbm.at[gid], buf, sem).wait()
            u = jnp.int32(0)
            for bit in _BITS:
                cand_u = u | bit
                cand_s = cand_u ^ _SIGN
                acc_ref[...] = jnp.zeros((n_lanes,), jnp.int32)

                @pl.loop(0, hw, step=n_lanes)
                def _(j):
                    v = buf[pl.ds(j, n_lanes)]
                    acc_ref[...] += (v >= cand_s).astype(jnp.int32)

                cnt = jnp.sum(acc_ref[...])
                u = jnp.where(cnt >= k_rank, cand_u, u)
            res_ref[...] = jnp.full((n_lanes,), u ^ _SIGN, jnp.int32)
            pltpu.async_copy(res_ref, o_hbm.at[gid], sem).wait()

    return select(keys)


def _pass2_kernel(z_ref, keys_ref, th_ref, wt_ref, bmlp_ref, out_ref,
                  gap_ref, *, ns, hw):
    s = pl.program_id(0)

    @pl.when(s == 0)
    def _init():
        gap_ref[...] = jnp.zeros(gap_ref.shape, gap_ref.dtype)

    mask = (keys_ref[...] > th_ref[:, 0:1]).astype(jnp.float32)
    gap_ref[...] += z_ref[...] * mask[:, None, :]

    @pl.when(s == ns - 1)
    def _finish():
        gap = jnp.sum(gap_ref[...], axis=2) * (1.0 / hw)
        out_ref[...] = jnp.dot(gap, wt_ref[...],
                               preferred_element_type=jnp.float32) \
            + bmlp_ref[...]


def kernel(z_in, w_phi, b_phi, W_mlp, b_mlp):
    b, c, h, w = z_in.shape
    hw = h * w
    k_rank = int(0.3 * hw)
    s_blk = 1024
    ns = hw // s_blk

    z_r = z_in.reshape(b, c, hw)
    w3 = w_phi.reshape(1, c, 1)
    bphi2 = jnp.broadcast_to(b_phi.reshape(1, 1), (1, 1)).astype(jnp.float32)
    wt = W_mlp.T
    bmlp2 = b_mlp.reshape(1, c)

    keys = pl.pallas_call(
        functools.partial(_pass1_kernel, s_blk=s_blk),
        grid=(ns,),
        in_specs=[
            pl.BlockSpec((b, c, s_blk), lambda s: (0, 0, s)),
            pl.BlockSpec((1, c, 1), lambda s: (0, 0, 0)),
            pl.BlockSpec((1, 1), lambda s: (0, 0)),
        ],
        out_specs=pl.BlockSpec((b, s_blk), lambda s: (0, s)),
        out_shape=jax.ShapeDtypeStruct((b, hw), jnp.int32),
        compiler_params=pltpu.CompilerParams(
            dimension_semantics=("arbitrary",),
        ),
    )(z_r, w3, bphi2)

    thresh = _sc_select(keys, k_rank, hw)

    return pl.pallas_call(
        functools.partial(_pass2_kernel, ns=ns, hw=hw),
        grid=(ns,),
        in_specs=[
            pl.BlockSpec((b, c, s_blk), lambda s: (0, 0, s)),
            pl.BlockSpec((b, s_blk), lambda s: (0, s)),
            pl.BlockSpec((b, 16), lambda s: (0, 0)),
            pl.BlockSpec((c, c), lambda s: (0, 0)),
            pl.BlockSpec((1, c), lambda s: (0, 0)),
        ],
        out_specs=pl.BlockSpec((b, c), lambda s: (0, 0)),
        out_shape=jax.ShapeDtypeStruct((b, c), jnp.float32),
        scratch_shapes=[
            pltpu.VMEM((b, c, s_blk), jnp.float32),
        ],
        compiler_params=pltpu.CompilerParams(
            dimension_semantics=("arbitrary",),
        ),
    )(z_r, keys, thresh, wt, bmlp2)


# speedup vs baseline: 1.6904x; 1.6904x over previous
import functools

import jax
import jax.numpy as jnp
import numpy as np
from jax.experimental import pallas as pl
from jax.experimental.pallas import tpu as pltpu

_SIGN = np.int32(-2**31)
_LOW31 = np.int32(2**31 - 1)
_BITS = [np.int32(-2**31)] + [np.int32(1 << i) for i in range(30, -1, -1)]


def _float_keys(x):
    b = jax.lax.bitcast_convert_type(x, jnp.int32)
    return jnp.where(b >= 0, b, b ^ _LOW31)


def _fused_kernel(z_ref, w_ref, bphi_ref, wt_ref, bmlp_ref, out_ref,
                  keys_ref, thresh_ref, gap_ref, mw_ref, *, ns, s_blk, k_rank, hw):
    p = pl.program_id(0)
    s = pl.program_id(1)
    b, c, _ = z_ref.shape

    @pl.when(p == 0)
    def _phase0():
        acc = z_ref[:, 0:8, :] * w_ref[:, 0:8, :]
        for r in range(1, c // 8):
            acc = acc + z_ref[:, 8 * r:8 * (r + 1), :] * w_ref[:, 8 * r:8 * (r + 1), :]
        xv = jnp.sum(acc, axis=1) + bphi_ref[...]
        keys_ref[:, pl.ds(s * s_blk, s_blk)] = _float_keys(xv)

    @pl.when(p == 1)
    def _phase1():
        @pl.when(s == 0)
        def _select():
            all_keys = keys_ref[...]
            u = jnp.zeros((all_keys.shape[0], 1), jnp.int32)
            for bit in _BITS:
                cand_u = u | bit
                cand_s = cand_u ^ _SIGN
                cnt = jnp.sum((all_keys >= cand_s).astype(jnp.int32),
                              axis=1, keepdims=True)
                u = jnp.where(cnt >= k_rank, cand_u, u)
            thresh_ref[...] = u ^ _SIGN
            gap_ref[...] = jnp.zeros(gap_ref.shape, gap_ref.dtype)

        keys = keys_ref[:, pl.ds(s * s_blk, s_blk)]
        mask = (keys > thresh_ref[...]).astype(jnp.float32)
        gap_ref[...] += z_ref[...] * mask[:, None, :]

        @pl.when(s == ns - 1)
        def _finish():
            gap = jnp.sum(gap_ref[...], axis=2) * (1.0 / hw)
            out_ref[...] = jnp.dot(gap, wt_ref[...],
                                   preferred_element_type=jnp.float32) \
                + bmlp_ref[...]


def kernel(z_in, w_phi, b_phi, W_mlp, b_mlp):
    b, c, h, w = z_in.shape
    hw = h * w
    k_rank = int(0.3 * hw)
    s_blk = 512
    ns = hw // s_blk

    z_r = z_in.reshape(b, c, hw)
    w3 = w_phi.reshape(1, c, 1)
    bphi2 = jnp.broadcast_to(b_phi.reshape(1, 1), (1, 1)).astype(jnp.float32)
    wt = W_mlp.T
    bmlp2 = b_mlp.reshape(1, c)

    grid = (2, ns)
    fn = functools.partial(_fused_kernel, ns=ns, s_blk=s_blk,
                           k_rank=k_rank, hw=hw)
    return pl.pallas_call(
        fn,
        grid=grid,
        in_specs=[
            pl.BlockSpec((b, c, s_blk), lambda p, s: (0, 0, s)),
            pl.BlockSpec((1, c, 1), lambda p, s: (0, 0, 0)),
            pl.BlockSpec((1, 1), lambda p, s: (0, 0)),
            pl.BlockSpec((c, c), lambda p, s: (0, 0)),
            pl.BlockSpec((1, c), lambda p, s: (0, 0)),
        ],
        out_specs=pl.BlockSpec((b, c), lambda p, s: (0, 0)),
        out_shape=jax.ShapeDtypeStruct((b, c), jnp.float32),
        scratch_shapes=[
            pltpu.VMEM((b, hw), jnp.int32),
            pltpu.VMEM((b, 1), jnp.int32),
            pltpu.VMEM((b, c, s_blk), jnp.float32),
            pltpu.VMEM((b, 8, s_blk), jnp.float32),
        ],
        compiler_params=pltpu.CompilerParams(
            dimension_semantics=("arbitrary", "arbitrary"),
        ),
    )(z_r, w3, bphi2, wt, bmlp2)


# speedup vs baseline: 1.7914x; 1.0598x over previous
import functools

import jax
import jax.numpy as jnp
import numpy as np
from jax.experimental import pallas as pl
from jax.experimental.pallas import tpu as pltpu

_SIGN = np.int32(-2**31)
_LOW31 = np.int32(2**31 - 1)
_BITS = [np.int32(-2**31)] + [np.int32(1 << i) for i in range(30, -1, -1)]


def _float_keys(x):
    b = jax.lax.bitcast_convert_type(x, jnp.int32)
    return jnp.where(b >= 0, b, b ^ _LOW31)


def _fused_kernel(z_ref, w_ref, bphi_ref, wt_ref, bmlp_ref, out_ref,
                  keys_ref, thresh_ref, gap_ref, mw_ref, *, ns, s_blk, k_rank, hw):
    p = pl.program_id(0)
    s = pl.program_id(1)
    b, c, _ = z_ref.shape

    @pl.when(p == 0)
    def _phase0():
        acc = z_ref[:, 0:8, :] * w_ref[:, 0:8, :]
        for r in range(1, c // 8):
            acc = acc + z_ref[:, 8 * r:8 * (r + 1), :] * w_ref[:, 8 * r:8 * (r + 1), :]
        xv = jnp.sum(acc, axis=1) + bphi_ref[...]
        keys_ref[:, pl.ds(s * s_blk, s_blk)] = _float_keys(xv)

    @pl.when(p == 1)
    def _phase1():
        @pl.when(s == 0)
        def _select():
            all_keys = keys_ref[...]
            u = jnp.zeros((all_keys.shape[0], 1), jnp.int32)
            for bit in _BITS:
                cand_u = u | bit
                cand_s = cand_u ^ _SIGN
                cnt = jnp.sum((all_keys >= cand_s).astype(jnp.int32),
                              axis=1, keepdims=True)
                u = jnp.where(cnt >= k_rank, cand_u, u)
            thresh_ref[...] = u ^ _SIGN
            gap_ref[...] = jnp.zeros(gap_ref.shape, gap_ref.dtype)
            mask_full = (all_keys > thresh_ref[...]).astype(jnp.float32)
            mw_ref[...] = jnp.broadcast_to(mask_full[:, None, :],
                                           mw_ref.shape)

        mg = mw_ref[:, :, pl.ds(s * s_blk, s_blk)]
        zg = z_ref[...].reshape(b, c // 8, 8, s_blk)
        prod = zg * mg[:, None, :, :]
        gap_ref[...] = (gap_ref[...].reshape(b, c // 8, 8, s_blk)
                        + prod).reshape(b, c, s_blk)

        @pl.when(s == ns - 1)
        def _finish():
            gap = jnp.sum(gap_ref[...], axis=2) * (1.0 / hw)
            out_ref[...] = jnp.dot(gap, wt_ref[...],
                                   preferred_element_type=jnp.float32) \
                + bmlp_ref[...]


def kernel(z_in, w_phi, b_phi, W_mlp, b_mlp):
    b, c, h, w = z_in.shape
    hw = h * w
    k_rank = int(0.3 * hw)
    s_blk = 1024
    ns = hw // s_blk

    z_r = z_in.reshape(b, c, hw)
    w3 = w_phi.reshape(1, c, 1)
    bphi2 = jnp.broadcast_to(b_phi.reshape(1, 1), (1, 1)).astype(jnp.float32)
    wt = W_mlp.T
    bmlp2 = b_mlp.reshape(1, c)

    grid = (2, ns)
    fn = functools.partial(_fused_kernel, ns=ns, s_blk=s_blk,
                           k_rank=k_rank, hw=hw)
    return pl.pallas_call(
        fn,
        grid=grid,
        in_specs=[
            pl.BlockSpec((b, c, s_blk), lambda p, s: (0, 0, s)),
            pl.BlockSpec((1, c, 1), lambda p, s: (0, 0, 0)),
            pl.BlockSpec((1, 1), lambda p, s: (0, 0)),
            pl.BlockSpec((c, c), lambda p, s: (0, 0)),
            pl.BlockSpec((1, c), lambda p, s: (0, 0)),
        ],
        out_specs=pl.BlockSpec((b, c), lambda p, s: (0, 0)),
        out_shape=jax.ShapeDtypeStruct((b, c), jnp.float32),
        scratch_shapes=[
            pltpu.VMEM((b, hw), jnp.int32),
            pltpu.VMEM((b, 1), jnp.int32),
            pltpu.VMEM((b, c, s_blk), jnp.float32),
            pltpu.VMEM((b, 8, hw), jnp.float32),
        ],
        compiler_params=pltpu.CompilerParams(
            dimension_semantics=("arbitrary", "arbitrary"),
        ),
    )(z_r, w3, bphi2, wt, bmlp2)
